# baseline (speedup 1.0000x reference)
import jax
import jax.numpy as jnp
from jax import lax
from jax.experimental import pallas as pl
from jax.experimental.pallas import tpu as pltpu
from jax.experimental.pallas import tpu_sc as plsc

_NSEG = 1024
_NCAND = 50000
_NS = 16
_NMERGE = 8
_SEGS_PER_MTILE = _NSEG // _NMERGE
_CHUNK = 3136
_PAD = _NS * _CHUNK
_GROUPS = _CHUNK // 16
_SENT = 2147483647


def _body(lp_hbm, cand_hbm, batch_hbm, out_hbm,
          idx_v, lp_v, seg_v, tmax_v, warg_v,
          mrg_f, mrg_i, widx_v, ones_v, zero_v,
          shf, shi, sem, gsem):
    s = lax.axis_index("s")
    base = jnp.minimum(s * _CHUNK, _NCAND - _CHUNK)

    cp_i = pltpu.async_copy(cand_hbm.at[pl.ds(base, _CHUNK)], idx_v, sem)
    cp_l = pltpu.async_copy(lp_hbm.at[pl.ds(base, _CHUNK)], lp_v, sem)

    def zb(k, c):
        zero_v[pl.ds(k * 16, 16)] = jnp.zeros((16,), jnp.float32)
        return c
    lax.fori_loop(0, _GROUPS, zb, 0)
    pltpu.sync_copy(zero_v, out_hbm.at[pl.ds(s * _CHUNK, _CHUNK)])

    def ib(k, c):
        tmax_v[pl.ds(k * 16, 16)] = jnp.full((16,), -jnp.inf, jnp.float32)
        warg_v[pl.ds(k * 16, 16)] = jnp.full((16,), _SENT, jnp.int32)
        return c
    lax.fori_loop(0, _NSEG // 16, ib, 0)

    cp_i.wait()
    cp_g = pltpu.async_copy(batch_hbm.at[idx_v], seg_v, gsem)
    cp_l.wait()
    cp_g.wait()

    iota16 = lax.iota(jnp.int32, 16)

    def g1(j, c):
        off = j * 16
        sg = seg_v[pl.ds(off, 16)]
        v = lp_v[pl.ds(off, 16)]
        ii = base + off + iota16
        cur0 = plsc.load_gather(tmax_v, [sg])
        carg0 = plsc.load_gather(warg_v, [sg])

        def cond(c2):
            cur, carg = c2
            return jnp.any((v > cur) | ((v == cur) & (ii < carg)))

        def bdy(c2):
            cur, carg = c2
            better = (v > cur) | ((v == cur) & (ii < carg))
            plsc.store_scatter(warg_v, [sg], ii, mask=better)
            carg2 = plsc.load_gather(warg_v, [sg])
            plsc.store_scatter(tmax_v, [sg], v, mask=carg2 == ii)
            cur2 = plsc.load_gather(tmax_v, [sg])
            return (cur2, carg2)

        lax.while_loop(cond, bdy, (cur0, carg0))
        return c
    lax.fori_loop(0, _GROUPS, g1, 0)

    pltpu.sync_copy(tmax_v, shf.at[pl.ds(s * _NSEG, _NSEG)])
    pltpu.sync_copy(warg_v, shi.at[pl.ds(s * _NSEG, _NSEG)])
    plsc.subcore_barrier()

    @pl.when(s < _NMERGE)
    def _merge():
        cps = []
        for r in range(_NS):
            src = pl.ds(r * _NSEG + s * _SEGS_PER_MTILE, _SEGS_PER_MTILE)
            dst = pl.ds(r * _SEGS_PER_MTILE, _SEGS_PER_MTILE)
            cps.append(pltpu.async_copy(shf.at[src], mrg_f.at[dst], sem))
            cps.append(pltpu.async_copy(shi.at[src], mrg_i.at[dst], sem))
        for cp in cps:
            cp.wait()
        for c8 in range(_SEGS_PER_MTILE // 16):
            av = mrg_f[pl.ds(c8 * 16, 16)]
            aa = mrg_i[pl.ds(c8 * 16, 16)]
            for r in range(1, _NS):
                bv = mrg_f[pl.ds(r * _SEGS_PER_MTILE + c8 * 16, 16)]
                ba = mrg_i[pl.ds(r * _SEGS_PER_MTILE + c8 * 16, 16)]
                take = (bv > av) | ((bv == av) & (ba < aa))
                av = jnp.where(take, bv, av)
                aa = jnp.where(take, ba, aa)
            aa = jnp.where(aa < _NCAND, aa, jnp.int32(_PAD - 1))
            widx_v[pl.ds(c8 * 16, 16)] = aa
            ones_v[pl.ds(c8 * 16, 16)] = jnp.ones((16,), jnp.float32)
        pltpu.async_copy(ones_v, out_hbm.at[widx_v], sem).wait()


_mesh = plsc.VectorSubcoreMesh(
    core_axis_name="c", subcore_axis_name="s", num_cores=1)

_call = pl.kernel(
    _body,
    out_type=jax.ShapeDtypeStruct((_PAD,), jnp.float32),
    mesh=_mesh,
    compiler_params=pltpu.CompilerParams(needs_layout_passes=False),
    scratch_types=[
        pltpu.VMEM((_CHUNK,), jnp.int32),
        pltpu.VMEM((_CHUNK,), jnp.float32),
        pltpu.VMEM((_CHUNK,), jnp.int32),
        pltpu.VMEM((_NSEG,), jnp.float32),
        pltpu.VMEM((_NSEG,), jnp.int32),
        pltpu.VMEM((_NS * _SEGS_PER_MTILE,), jnp.float32),
        pltpu.VMEM((_NS * _SEGS_PER_MTILE,), jnp.int32),
        pltpu.VMEM((_SEGS_PER_MTILE,), jnp.int32),
        pltpu.VMEM((_SEGS_PER_MTILE,), jnp.float32),
        pltpu.VMEM((_CHUNK,), jnp.float32),
        pltpu.VMEM_SHARED((_NS * _NSEG,), jnp.float32),
        pltpu.VMEM_SHARED((_NS * _NSEG,), jnp.int32),
        pltpu.SemaphoreType.DMA,
        pltpu.SemaphoreType.DMA,
    ],
)


def kernel(log_probs, batch, candidate_idxs):
    winners = _call(log_probs, candidate_idxs, batch)[:_NCAND]
    return (log_probs, winners)


# speedup vs baseline: 1.3169x; 1.3169x over previous
import jax
import jax.numpy as jnp
from jax import lax
from jax.experimental import pallas as pl
from jax.experimental.pallas import tpu as pltpu
from jax.experimental.pallas import tpu_sc as plsc

_NSEG = 1024
_NCAND = 50000
_NNODES = 100000
_NS = 16
_NMERGE = 8
_SEGS_PER_MTILE = _NSEG // _NMERGE
_CHUNK = 3136
_PAD = _NS * _CHUNK
_GROUPS = _CHUNK // 16
_SENT = 2147483647
_NNODES_PAD = 100352
_BCHUNK = _NNODES_PAD // 8
_GCHUNK = 128
_NGFULL = _CHUNK // _GCHUNK
_GTAIL = _CHUNK - _NGFULL * _GCHUNK


def _body(lp_hbm, cand_hbm, batch_hbm, out_hbm,
          idx_v, lp_v, seg_v, tmax_v, warg_v,
          mrg_f, mrg_i, widx_v, ones_v, zero_v,
          shb, shf, shi, sem, gsem):
    s = lax.axis_index("s")
    base = jnp.minimum(s * _CHUNK, _NCAND - _CHUNK)

    cp_i = pltpu.async_copy(cand_hbm.at[pl.ds(base, _CHUNK)], idx_v, sem)
    cp_l = pltpu.async_copy(lp_hbm.at[pl.ds(base, _CHUNK)], lp_v, sem)

    @pl.when(s < 8)
    def _stage_batch():
        pltpu.sync_copy(batch_hbm.at[pl.ds(s * _BCHUNK, _BCHUNK)],
                        shb.at[pl.ds(s * _BCHUNK, _BCHUNK)])

    def zb(k, c):
        zero_v[pl.ds(k * 16, 16)] = jnp.zeros((16,), jnp.float32)
        return c
    lax.fori_loop(0, _GROUPS, zb, 0)
    pltpu.sync_copy(zero_v, out_hbm.at[pl.ds(s * _CHUNK, _CHUNK)])

    def ib(k, c):
        tmax_v[pl.ds(k * 16, 16)] = jnp.full((16,), -jnp.inf, jnp.float32)
        warg_v[pl.ds(k * 16, 16)] = jnp.full((16,), _SENT, jnp.int32)
        return c
    lax.fori_loop(0, _NSEG // 16, ib, 0)

    cp_i.wait()
    cp_l.wait()
    plsc.subcore_barrier()

    gcps = [
        pltpu.async_copy(shb.at[idx_v.at[pl.ds(k * _GCHUNK, _GCHUNK)]],
                         seg_v.at[pl.ds(k * _GCHUNK, _GCHUNK)], gsem)
        for k in range(_NGFULL)
    ]
    gcps.append(
        pltpu.async_copy(shb.at[idx_v.at[pl.ds(_NGFULL * _GCHUNK, _GTAIL)]],
                         seg_v.at[pl.ds(_NGFULL * _GCHUNK, _GTAIL)], gsem))
    for cp in gcps:
        cp.wait()

    iota16 = lax.iota(jnp.int32, 16)
    _W = 4

    def g1(j, c):
        offs = [j * 16 * _W + q * 16 for q in range(_W)]
        sgs = [seg_v[pl.ds(o, 16)] for o in offs]
        vs = [lp_v[pl.ds(o, 16)] for o in offs]
        iis = [base + o + iota16 for o in offs]
        cur0 = [plsc.load_gather(tmax_v, [sg]) for sg in sgs]
        carg0 = [plsc.load_gather(warg_v, [sg]) for sg in sgs]

        def cond(c2):
            cur, carg = c2
            m = [(vs[q] > cur[q]) | ((vs[q] == cur[q]) & (iis[q] < carg[q]))
                 for q in range(_W)]
            acc = m[0]
            for q in range(1, _W):
                acc = acc | m[q]
            return jnp.any(acc)

        def bdy(c2):
            cur, carg = c2
            for q in range(_W):
                better = ((vs[q] > cur[q])
                          | ((vs[q] == cur[q]) & (iis[q] < carg[q])))
                plsc.store_scatter(warg_v, [sgs[q]], iis[q], mask=better)
            carg2 = [plsc.load_gather(warg_v, [sg]) for sg in sgs]
            for q in range(_W):
                plsc.store_scatter(tmax_v, [sgs[q]], vs[q],
                                   mask=carg2[q] == iis[q])
            cur2 = [plsc.load_gather(tmax_v, [sg]) for sg in sgs]
            return (cur2, carg2)

        lax.while_loop(cond, bdy, (cur0, carg0))
        return c
    lax.fori_loop(0, _GROUPS // _W, g1, 0)

    pltpu.sync_copy(tmax_v, shf.at[pl.ds(s * _NSEG, _NSEG)])
    pltpu.sync_copy(warg_v, shi.at[pl.ds(s * _NSEG, _NSEG)])
    plsc.subcore_barrier()

    @pl.when(s < _NMERGE)
    def _merge():
        cps = []
        for r in range(_NS):
            src = pl.ds(r * _NSEG + s * _SEGS_PER_MTILE, _SEGS_PER_MTILE)
            dst = pl.ds(r * _SEGS_PER_MTILE, _SEGS_PER_MTILE)
            cps.append(pltpu.async_copy(shf.at[src], mrg_f.at[dst], sem))
            cps.append(pltpu.async_copy(shi.at[src], mrg_i.at[dst], sem))
        for cp in cps:
            cp.wait()
        for c8 in range(_SEGS_PER_MTILE // 16):
            av = mrg_f[pl.ds(c8 * 16, 16)]
            aa = mrg_i[pl.ds(c8 * 16, 16)]
            for r in range(1, _NS):
                bv = mrg_f[pl.ds(r * _SEGS_PER_MTILE + c8 * 16, 16)]
                ba = mrg_i[pl.ds(r * _SEGS_PER_MTILE + c8 * 16, 16)]
                take = (bv > av) | ((bv == av) & (ba < aa))
                av = jnp.where(take, bv, av)
                aa = jnp.where(take, ba, aa)
            aa = jnp.where(aa < _NCAND, aa, jnp.int32(_PAD - 1))
            widx_v[pl.ds(c8 * 16, 16)] = aa
            ones_v[pl.ds(c8 * 16, 16)] = jnp.ones((16,), jnp.float32)
        pltpu.async_copy(ones_v, out_hbm.at[widx_v], sem).wait()


_mesh = plsc.VectorSubcoreMesh(
    core_axis_name="c", subcore_axis_name="s", num_cores=1)

_call = pl.kernel(
    _body,
    out_type=jax.ShapeDtypeStruct((_PAD,), jnp.float32),
    mesh=_mesh,
    compiler_params=pltpu.CompilerParams(needs_layout_passes=False),
    scratch_types=[
        pltpu.VMEM((_CHUNK,), jnp.int32),
        pltpu.VMEM((_CHUNK,), jnp.float32),
        pltpu.VMEM((_CHUNK,), jnp.int32),
        pltpu.VMEM((_NSEG,), jnp.float32),
        pltpu.VMEM((_NSEG,), jnp.int32),
        pltpu.VMEM((_NS * _SEGS_PER_MTILE,), jnp.float32),
        pltpu.VMEM((_NS * _SEGS_PER_MTILE,), jnp.int32),
        pltpu.VMEM((_SEGS_PER_MTILE,), jnp.int32),
        pltpu.VMEM((_SEGS_PER_MTILE,), jnp.float32),
        pltpu.VMEM((_CHUNK,), jnp.float32),
        pltpu.VMEM_SHARED((_NNODES_PAD,), jnp.int32),
        pltpu.VMEM_SHARED((_NS * _NSEG,), jnp.float32),
        pltpu.VMEM_SHARED((_NS * _NSEG,), jnp.int32),
        pltpu.SemaphoreType.DMA,
        pltpu.SemaphoreType.DMA,
    ],
)


def kernel(log_probs, batch, candidate_idxs):
    batch_pad = jnp.concatenate(
        [batch, jnp.zeros((_NNODES_PAD - _NNODES,), jnp.int32)])
    winners = _call(log_probs, candidate_idxs, batch_pad)[:_NCAND]
    return (log_probs, winners)


# speedup vs baseline: 1.3183x; 1.0010x over previous
import jax
import jax.numpy as jnp
from jax import lax
from jax.experimental import pallas as pl
from jax.experimental.pallas import tpu as pltpu
from jax.experimental.pallas import tpu_sc as plsc

_NSEG = 1024
_NCAND = 50000
_NNODES = 100000
_NS = 16
_NMERGE = 8
_SEGS_PER_MTILE = _NSEG // _NMERGE
_CHUNK = 3200
_PAD = _NS * _CHUNK
_GROUPS = _CHUNK // 16
_SENT = 2147483647
_NNODES_PAD = 100352
_BCHUNK = _NNODES_PAD // 8
_GCHUNK = 128
_NGFULL = _CHUNK // _GCHUNK


def _body(lp_hbm, cand_hbm, batch_hbm, out_hbm,
          idx_v, lp_v, seg_v, tmax_v, warg_v,
          mrg_f, mrg_i, widx_v, ones_v, zero_v,
          shb, shf, shi, sem, gsem):
    s = lax.axis_index("s")
    base = jnp.minimum(s * _CHUNK, _NCAND - _CHUNK)

    cp_i = pltpu.async_copy(cand_hbm.at[pl.ds(base, _CHUNK)], idx_v, sem)
    cp_l = pltpu.async_copy(lp_hbm.at[pl.ds(base, _CHUNK)], lp_v, sem)

    @pl.when(s < 8)
    def _stage_batch():
        pltpu.sync_copy(batch_hbm.at[pl.ds(s * _BCHUNK, _BCHUNK)],
                        shb.at[pl.ds(s * _BCHUNK, _BCHUNK)])

    def zb(k, c):
        zero_v[pl.ds(k * 16, 16)] = jnp.zeros((16,), jnp.float32)
        return c
    lax.fori_loop(0, _GROUPS, zb, 0)
    pltpu.sync_copy(zero_v, out_hbm.at[pl.ds(s * _CHUNK, _CHUNK)])

    def ib(k, c):
        tmax_v[pl.ds(k * 16, 16)] = jnp.full((16,), -jnp.inf, jnp.float32)
        warg_v[pl.ds(k * 16, 16)] = jnp.full((16,), _SENT, jnp.int32)
        return c
    lax.fori_loop(0, _NSEG // 16, ib, 0)

    cp_i.wait()
    cp_l.wait()
    plsc.subcore_barrier()

    gcps = [
        pltpu.async_copy(shb.at[idx_v.at[pl.ds(k * _GCHUNK, _GCHUNK)]],
                         seg_v.at[pl.ds(k * _GCHUNK, _GCHUNK)], gsem)
        for k in range(_NGFULL)
    ]
    for cp in gcps:
        cp.wait()

    iota16 = lax.iota(jnp.int32, 16)
    _W = 8

    def g1(j, c):
        offs = [j * 16 * _W + q * 16 for q in range(_W)]
        sgs = [seg_v[pl.ds(o, 16)] for o in offs]
        vs = [lp_v[pl.ds(o, 16)] for o in offs]
        iis = [base + o + iota16 for o in offs]
        cur0 = [plsc.load_gather(tmax_v, [sg]) for sg in sgs]
        carg0 = [plsc.load_gather(warg_v, [sg]) for sg in sgs]

        def cond(c2):
            cur, carg = c2
            m = [(vs[q] > cur[q]) | ((vs[q] == cur[q]) & (iis[q] < carg[q]))
                 for q in range(_W)]
            acc = m[0]
            for q in range(1, _W):
                acc = acc | m[q]
            return jnp.any(acc)

        def bdy(c2):
            cur, carg = c2
            for q in range(_W):
                better = ((vs[q] > cur[q])
                          | ((vs[q] == cur[q]) & (iis[q] < carg[q])))
                plsc.store_scatter(warg_v, [sgs[q]], iis[q], mask=better)
            carg2 = [plsc.load_gather(warg_v, [sg]) for sg in sgs]
            for q in range(_W):
                plsc.store_scatter(tmax_v, [sgs[q]], vs[q],
                                   mask=carg2[q] == iis[q])
            cur2 = [plsc.load_gather(tmax_v, [sg]) for sg in sgs]
            return (cur2, carg2)

        lax.while_loop(cond, bdy, (cur0, carg0))
        return c
    lax.fori_loop(0, _GROUPS // _W, g1, 0)

    pltpu.sync_copy(tmax_v, shf.at[pl.ds(s * _NSEG, _NSEG)])
    pltpu.sync_copy(warg_v, shi.at[pl.ds(s * _NSEG, _NSEG)])
    plsc.subcore_barrier()

    @pl.when(s < _NMERGE)
    def _merge():
        cps = []
        for r in range(_NS):
            src = pl.ds(r * _NSEG + s * _SEGS_PER_MTILE, _SEGS_PER_MTILE)
            dst = pl.ds(r * _SEGS_PER_MTILE, _SEGS_PER_MTILE)
            cps.append(pltpu.async_copy(shf.at[src], mrg_f.at[dst], sem))
            cps.append(pltpu.async_copy(shi.at[src], mrg_i.at[dst], sem))
        for cp in cps:
            cp.wait()
        for c8 in range(_SEGS_PER_MTILE // 16):
            av = mrg_f[pl.ds(c8 * 16, 16)]
            aa = mrg_i[pl.ds(c8 * 16, 16)]
            for r in range(1, _NS):
                bv = mrg_f[pl.ds(r * _SEGS_PER_MTILE + c8 * 16, 16)]
                ba = mrg_i[pl.ds(r * _SEGS_PER_MTILE + c8 * 16, 16)]
                take = (bv > av) | ((bv == av) & (ba < aa))
                av = jnp.where(take, bv, av)
                aa = jnp.where(take, ba, aa)
            aa = jnp.where(aa < _NCAND, aa, jnp.int32(_PAD - 1))
            widx_v[pl.ds(c8 * 16, 16)] = aa
            ones_v[pl.ds(c8 * 16, 16)] = jnp.ones((16,), jnp.float32)
        pltpu.async_copy(ones_v, out_hbm.at[widx_v], sem).wait()


_mesh = plsc.VectorSubcoreMesh(
    core_axis_name="c", subcore_axis_name="s", num_cores=1)

_call = pl.kernel(
    _body,
    out_type=jax.ShapeDtypeStruct((_PAD,), jnp.float32),
    mesh=_mesh,
    compiler_params=pltpu.CompilerParams(needs_layout_passes=False),
    scratch_types=[
        pltpu.VMEM((_CHUNK,), jnp.int32),
        pltpu.VMEM((_CHUNK,), jnp.float32),
        pltpu.VMEM((_CHUNK,), jnp.int32),
        pltpu.VMEM((_NSEG,), jnp.float32),
        pltpu.VMEM((_NSEG,), jnp.int32),
        pltpu.VMEM((_NS * _SEGS_PER_MTILE,), jnp.float32),
        pltpu.VMEM((_NS * _SEGS_PER_MTILE,), jnp.int32),
        pltpu.VMEM((_SEGS_PER_MTILE,), jnp.int32),
        pltpu.VMEM((_SEGS_PER_MTILE,), jnp.float32),
        pltpu.VMEM((_CHUNK,), jnp.float32),
        pltpu.VMEM_SHARED((_NNODES_PAD,), jnp.int32),
        pltpu.VMEM_SHARED((_NS * _NSEG,), jnp.float32),
        pltpu.VMEM_SHARED((_NS * _NSEG,), jnp.int32),
        pltpu.SemaphoreType.DMA,
        pltpu.SemaphoreType.DMA,
    ],
)


def kernel(log_probs, batch, candidate_idxs):
    batch_pad = jnp.concatenate(
        [batch, jnp.zeros((_NNODES_PAD - _NNODES,), jnp.int32)])
    winners = _call(log_probs, candidate_idxs, batch_pad)[:_NCAND]
    return (log_probs, winners)


# speedup vs baseline: 1.3273x; 1.0068x over previous
import jax
import jax.numpy as jnp
from jax import lax
from jax.experimental import pallas as pl
from jax.experimental.pallas import tpu as pltpu
from jax.experimental.pallas import tpu_sc as plsc

_NSEG = 1024
_NCAND = 50000
_NNODES = 100000
_NS = 16
_NMERGE = 8
_SEGS_PER_MTILE = _NSEG // _NMERGE
_CHUNK = 3200
_PAD = _NS * _CHUNK
_GROUPS = _CHUNK // 16
_SENT = 2147483647
_NNODES_PAD = 100352
_BCHUNK = _NNODES_PAD // 8


def _body(lp_hbm, cand_hbm, batch_hbm, out_hbm,
          idx_v, lp_v, seg_v, tmax_v, warg_v,
          mrg_f, mrg_i, widx_v, ones_v, zero_v,
          shb, shf, shi, sem, gsem):
    s = lax.axis_index("s")
    base = jnp.minimum(s * _CHUNK, _NCAND - _CHUNK)

    cp_i = pltpu.async_copy(cand_hbm.at[pl.ds(base, _CHUNK)], idx_v, sem)
    cp_l = pltpu.async_copy(lp_hbm.at[pl.ds(base, _CHUNK)], lp_v, sem)

    @pl.when(s < 8)
    def _stage_batch():
        pltpu.sync_copy(batch_hbm.at[pl.ds(s * _BCHUNK, _BCHUNK)],
                        shb.at[pl.ds(s * _BCHUNK, _BCHUNK)])

    def zb(k, c):
        zero_v[pl.ds(k * 16, 16)] = jnp.zeros((16,), jnp.float32)
        return c
    lax.fori_loop(0, _GROUPS, zb, 0)
    pltpu.sync_copy(zero_v, out_hbm.at[pl.ds(s * _CHUNK, _CHUNK)])

    def ib(k, c):
        tmax_v[pl.ds(k * 16, 16)] = jnp.full((16,), -jnp.inf, jnp.float32)
        warg_v[pl.ds(k * 16, 16)] = jnp.full((16,), _SENT, jnp.int32)
        return c
    lax.fori_loop(0, _NSEG // 16, ib, 0)

    cp_i.wait()
    cp_l.wait()
    plsc.subcore_barrier()

    pltpu.async_copy(shb.at[idx_v], seg_v, gsem).wait()

    iota16 = lax.iota(jnp.int32, 16)
    _W = 8

    def g1(j, c):
        offs = [j * 16 * _W + q * 16 for q in range(_W)]
        sgs = [seg_v[pl.ds(o, 16)] for o in offs]
        vs = [lp_v[pl.ds(o, 16)] for o in offs]
        iis = [base + o + iota16 for o in offs]
        cur0 = [plsc.load_gather(tmax_v, [sg]) for sg in sgs]
        carg0 = [plsc.load_gather(warg_v, [sg]) for sg in sgs]

        def cond(c2):
            cur, carg = c2
            m = [(vs[q] > cur[q]) | ((vs[q] == cur[q]) & (iis[q] < carg[q]))
                 for q in range(_W)]
            acc = m[0]
            for q in range(1, _W):
                acc = acc | m[q]
            return jnp.any(acc)

        def bdy(c2):
            cur, carg = c2
            for q in range(_W):
                better = ((vs[q] > cur[q])
                          | ((vs[q] == cur[q]) & (iis[q] < carg[q])))
                plsc.store_scatter(warg_v, [sgs[q]], iis[q], mask=better)
            carg2 = [plsc.load_gather(warg_v, [sg]) for sg in sgs]
            for q in range(_W):
                plsc.store_scatter(tmax_v, [sgs[q]], vs[q],
                                   mask=carg2[q] == iis[q])
            cur2 = [plsc.load_gather(tmax_v, [sg]) for sg in sgs]
            return (cur2, carg2)

        lax.while_loop(cond, bdy, (cur0, carg0))
        return c
    lax.fori_loop(0, _GROUPS // _W, g1, 0)

    pltpu.sync_copy(tmax_v, shf.at[pl.ds(s * _NSEG, _NSEG)])
    pltpu.sync_copy(warg_v, shi.at[pl.ds(s * _NSEG, _NSEG)])
    plsc.subcore_barrier()

    @pl.when(s < _NMERGE)
    def _merge():
        cps = []
        for r in range(_NS):
            src = pl.ds(r * _NSEG + s * _SEGS_PER_MTILE, _SEGS_PER_MTILE)
            dst = pl.ds(r * _SEGS_PER_MTILE, _SEGS_PER_MTILE)
            cps.append(pltpu.async_copy(shf.at[src], mrg_f.at[dst], sem))
            cps.append(pltpu.async_copy(shi.at[src], mrg_i.at[dst], sem))
        for cp in cps:
            cp.wait()
        for c8 in range(_SEGS_PER_MTILE // 16):
            av = mrg_f[pl.ds(c8 * 16, 16)]
            aa = mrg_i[pl.ds(c8 * 16, 16)]
            for r in range(1, _NS):
                bv = mrg_f[pl.ds(r * _SEGS_PER_MTILE + c8 * 16, 16)]
                ba = mrg_i[pl.ds(r * _SEGS_PER_MTILE + c8 * 16, 16)]
                take = (bv > av) | ((bv == av) & (ba < aa))
                av = jnp.where(take, bv, av)
                aa = jnp.where(take, ba, aa)
            aa = jnp.where(aa < _NCAND, aa, jnp.int32(_PAD - 1))
            widx_v[pl.ds(c8 * 16, 16)] = aa
            ones_v[pl.ds(c8 * 16, 16)] = jnp.ones((16,), jnp.float32)
        pltpu.async_copy(ones_v, out_hbm.at[widx_v], sem).wait()


_mesh = plsc.VectorSubcoreMesh(
    core_axis_name="c", subcore_axis_name="s", num_cores=1)

_call = pl.kernel(
    _body,
    out_type=jax.ShapeDtypeStruct((_PAD,), jnp.float32),
    mesh=_mesh,
    compiler_params=pltpu.CompilerParams(needs_layout_passes=False),
    scratch_types=[
        pltpu.VMEM((_CHUNK,), jnp.int32),
        pltpu.VMEM((_CHUNK,), jnp.float32),
        pltpu.VMEM((_CHUNK,), jnp.int32),
        pltpu.VMEM((_NSEG,), jnp.float32),
        pltpu.VMEM((_NSEG,), jnp.int32),
        pltpu.VMEM((_NS * _SEGS_PER_MTILE,), jnp.float32),
        pltpu.VMEM((_NS * _SEGS_PER_MTILE,), jnp.int32),
        pltpu.VMEM((_SEGS_PER_MTILE,), jnp.int32),
        pltpu.VMEM((_SEGS_PER_MTILE,), jnp.float32),
        pltpu.VMEM((_CHUNK,), jnp.float32),
        pltpu.VMEM_SHARED((_NNODES_PAD,), jnp.int32),
        pltpu.VMEM_SHARED((_NS * _NSEG,), jnp.float32),
        pltpu.VMEM_SHARED((_NS * _NSEG,), jnp.int32),
        pltpu.SemaphoreType.DMA,
        pltpu.SemaphoreType.DMA,
    ],
)


def kernel(log_probs, batch, candidate_idxs):
    batch_pad = jnp.concatenate(
        [batch, jnp.zeros((_NNODES_PAD - _NNODES,), jnp.int32)])
    winners = _call(log_probs, candidate_idxs, batch_pad)[:_NCAND]
    return (log_probs, winners)


# speedup vs baseline: 1.3276x; 1.0003x over previous
import jax
import jax.numpy as jnp
from jax import lax
from jax.experimental import pallas as pl
from jax.experimental.pallas import tpu as pltpu
from jax.experimental.pallas import tpu_sc as plsc

_NSEG = 1024
_NCAND = 50000
_NNODES = 100000
_NS = 16
_NMERGE = 8
_SEGS_PER_MTILE = _NSEG // _NMERGE
_CHUNK = 3200
_PAD = _NS * _CHUNK
_GROUPS = _CHUNK // 16
_SENT = 2147483647
_NNODES_PAD = 100352
_BCHUNK = _NNODES_PAD // _NS


def _body(lp_hbm, cand_hbm, batch_hbm, out_hbm,
          idx_v, lp_v, seg_v, tmax_v, warg_v,
          mrg_f, mrg_i, widx_v, ones_v, zero_v,
          shb, shf, shi, sem, gsem):
    s = lax.axis_index("s")
    base = jnp.minimum(s * _CHUNK, _NCAND - _CHUNK)

    cp_i = pltpu.async_copy(cand_hbm.at[pl.ds(base, _CHUNK)], idx_v, sem)
    cp_l = pltpu.async_copy(lp_hbm.at[pl.ds(base, _CHUNK)], lp_v, sem)

    pltpu.sync_copy(batch_hbm.at[pl.ds(s * _BCHUNK, _BCHUNK)],
                    shb.at[pl.ds(s * _BCHUNK, _BCHUNK)])

    def zb(k, c):
        zero_v[pl.ds(k * 16, 16)] = jnp.zeros((16,), jnp.float32)
        return c
    lax.fori_loop(0, _GROUPS, zb, 0)
    pltpu.sync_copy(zero_v, out_hbm.at[pl.ds(s * _CHUNK, _CHUNK)])

    def ib(k, c):
        tmax_v[pl.ds(k * 16, 16)] = jnp.full((16,), -jnp.inf, jnp.float32)
        warg_v[pl.ds(k * 16, 16)] = jnp.full((16,), _SENT, jnp.int32)
        return c
    lax.fori_loop(0, _NSEG // 16, ib, 0)

    cp_i.wait()
    cp_l.wait()
    plsc.subcore_barrier()

    pltpu.async_copy(shb.at[idx_v], seg_v, gsem).wait()

    iota16 = lax.iota(jnp.int32, 16)
    _W = 8

    def g1(j, c):
        offs = [j * 16 * _W + q * 16 for q in range(_W)]
        sgs = [seg_v[pl.ds(o, 16)] for o in offs]
        vs = [lp_v[pl.ds(o, 16)] for o in offs]
        iis = [base + o + iota16 for o in offs]
        cur0 = [plsc.load_gather(tmax_v, [sg]) for sg in sgs]
        carg0 = [plsc.load_gather(warg_v, [sg]) for sg in sgs]

        def cond(c2):
            cur, carg = c2
            m = [(vs[q] > cur[q]) | ((vs[q] == cur[q]) & (iis[q] < carg[q]))
                 for q in range(_W)]
            acc = m[0]
            for q in range(1, _W):
                acc = acc | m[q]
            return jnp.any(acc)

        def bdy(c2):
            cur, carg = c2
            for q in range(_W):
                better = ((vs[q] > cur[q])
                          | ((vs[q] == cur[q]) & (iis[q] < carg[q])))
                plsc.store_scatter(warg_v, [sgs[q]], iis[q], mask=better)
            carg2 = [plsc.load_gather(warg_v, [sg]) for sg in sgs]
            for q in range(_W):
                plsc.store_scatter(tmax_v, [sgs[q]], vs[q],
                                   mask=carg2[q] == iis[q])
            cur2 = [plsc.load_gather(tmax_v, [sg]) for sg in sgs]
            return (cur2, carg2)

        lax.while_loop(cond, bdy, (cur0, carg0))
        return c
    lax.fori_loop(0, _GROUPS // _W, g1, 0)

    pltpu.sync_copy(tmax_v, shf.at[pl.ds(s * _NSEG, _NSEG)])
    pltpu.sync_copy(warg_v, shi.at[pl.ds(s * _NSEG, _NSEG)])
    plsc.subcore_barrier()

    @pl.when(s < _NMERGE)
    def _merge():
        cps = []
        for r in range(_NS):
            src = pl.ds(r * _NSEG + s * _SEGS_PER_MTILE, _SEGS_PER_MTILE)
            dst = pl.ds(r * _SEGS_PER_MTILE, _SEGS_PER_MTILE)
            cps.append(pltpu.async_copy(shf.at[src], mrg_f.at[dst], sem))
            cps.append(pltpu.async_copy(shi.at[src], mrg_i.at[dst], sem))
        for cp in cps:
            cp.wait()
        for c8 in range(_SEGS_PER_MTILE // 16):
            av = mrg_f[pl.ds(c8 * 16, 16)]
            aa = mrg_i[pl.ds(c8 * 16, 16)]
            for r in range(1, _NS):
                bv = mrg_f[pl.ds(r * _SEGS_PER_MTILE + c8 * 16, 16)]
                ba = mrg_i[pl.ds(r * _SEGS_PER_MTILE + c8 * 16, 16)]
                take = (bv > av) | ((bv == av) & (ba < aa))
                av = jnp.where(take, bv, av)
                aa = jnp.where(take, ba, aa)
            aa = jnp.where(aa < _NCAND, aa, jnp.int32(_PAD - 1))
            widx_v[pl.ds(c8 * 16, 16)] = aa
            ones_v[pl.ds(c8 * 16, 16)] = jnp.ones((16,), jnp.float32)
        pltpu.async_copy(ones_v, out_hbm.at[widx_v], sem).wait()


_mesh = plsc.VectorSubcoreMesh(
    core_axis_name="c", subcore_axis_name="s", num_cores=1)

_call = pl.kernel(
    _body,
    out_type=jax.ShapeDtypeStruct((_PAD,), jnp.float32),
    mesh=_mesh,
    compiler_params=pltpu.CompilerParams(needs_layout_passes=False),
    scratch_types=[
        pltpu.VMEM((_CHUNK,), jnp.int32),
        pltpu.VMEM((_CHUNK,), jnp.float32),
        pltpu.VMEM((_CHUNK,), jnp.int32),
        pltpu.VMEM((_NSEG,), jnp.float32),
        pltpu.VMEM((_NSEG,), jnp.int32),
        pltpu.VMEM((_NS * _SEGS_PER_MTILE,), jnp.float32),
        pltpu.VMEM((_NS * _SEGS_PER_MTILE,), jnp.int32),
        pltpu.VMEM((_SEGS_PER_MTILE,), jnp.int32),
        pltpu.VMEM((_SEGS_PER_MTILE,), jnp.float32),
        pltpu.VMEM((_CHUNK,), jnp.float32),
        pltpu.VMEM_SHARED((_NNODES_PAD,), jnp.int32),
        pltpu.VMEM_SHARED((_NS * _NSEG,), jnp.float32),
        pltpu.VMEM_SHARED((_NS * _NSEG,), jnp.int32),
        pltpu.SemaphoreType.DMA,
        pltpu.SemaphoreType.DMA,
    ],
)


def kernel(log_probs, batch, candidate_idxs):
    batch_pad = jnp.concatenate(
        [batch, jnp.zeros((_NNODES_PAD - _NNODES,), jnp.int32)])
    winners = _call(log_probs, candidate_idxs, batch_pad)[:_NCAND]
    return (log_probs, winners)


# speedup vs baseline: 1.3378x; 1.0076x over previous
import jax
import jax.numpy as jnp
from jax import lax
from jax.experimental import pallas as pl
from jax.experimental.pallas import tpu as pltpu
from jax.experimental.pallas import tpu_sc as plsc

_NSEG = 1024
_NCAND = 50000
_NNODES = 100000
_NS = 16
_NMERGE = 8
_SEGS_PER_MTILE = _NSEG // _NMERGE
_CHUNK = 3200
_PAD = _NS * _CHUNK
_GROUPS = _CHUNK // 16
_SENT = 2147483647
_NNODES_PAD = 100352
_BCHUNK = _NNODES_PAD // _NS


def _body(lp_hbm, cand_hbm, batch_hbm, out_hbm,
          idx_v, lp_v, seg_v, tmax_v, warg_v,
          mrg_f, mrg_i, widx_v, ones_v, zero_v,
          shb, shf, shi, sem, gsem):
    s = lax.axis_index("s")
    base = jnp.minimum(s * _CHUNK, _NCAND - _CHUNK)

    cp_i = pltpu.async_copy(cand_hbm.at[pl.ds(base, _CHUNK)], idx_v, sem)
    cp_l = pltpu.async_copy(lp_hbm.at[pl.ds(base, _CHUNK)], lp_v, sem)

    pltpu.sync_copy(batch_hbm.at[pl.ds(s * _BCHUNK, _BCHUNK)],
                    shb.at[pl.ds(s * _BCHUNK, _BCHUNK)])

    def zb(k, c):
        zero_v[pl.ds(k * 16, 16)] = jnp.zeros((16,), jnp.float32)
        return c
    lax.fori_loop(0, _GROUPS, zb, 0)
    cp_z = pltpu.async_copy(zero_v, out_hbm.at[pl.ds(s * _CHUNK, _CHUNK)],
                            sem)

    def ib(k, c):
        tmax_v[pl.ds(k * 16, 16)] = jnp.full((16,), -jnp.inf, jnp.float32)
        warg_v[pl.ds(k * 16, 16)] = jnp.full((16,), _SENT, jnp.int32)
        return c
    lax.fori_loop(0, _NSEG // 16, ib, 0)

    cp_i.wait()
    cp_l.wait()
    plsc.subcore_barrier()

    pltpu.async_copy(shb.at[idx_v], seg_v, gsem).wait()

    iota16 = lax.iota(jnp.int32, 16)
    _W = 8

    def g1(j, c):
        offs = [j * 16 * _W + q * 16 for q in range(_W)]
        sgs = [seg_v[pl.ds(o, 16)] for o in offs]
        vs = [lp_v[pl.ds(o, 16)] for o in offs]
        iis = [base + o + iota16 for o in offs]
        cur0 = [plsc.load_gather(tmax_v, [sg]) for sg in sgs]
        carg0 = [plsc.load_gather(warg_v, [sg]) for sg in sgs]

        def cond(c2):
            cur, carg = c2
            m = [(vs[q] > cur[q]) | ((vs[q] == cur[q]) & (iis[q] < carg[q]))
                 for q in range(_W)]
            acc = m[0]
            for q in range(1, _W):
                acc = acc | m[q]
            return jnp.any(acc)

        def bdy(c2):
            cur, carg = c2
            for q in range(_W):
                better = ((vs[q] > cur[q])
                          | ((vs[q] == cur[q]) & (iis[q] < carg[q])))
                plsc.store_scatter(warg_v, [sgs[q]], iis[q], mask=better)
            carg2 = [plsc.load_gather(warg_v, [sg]) for sg in sgs]
            for q in range(_W):
                plsc.store_scatter(tmax_v, [sgs[q]], vs[q],
                                   mask=carg2[q] == iis[q])
            cur2 = [plsc.load_gather(tmax_v, [sg]) for sg in sgs]
            return (cur2, carg2)

        lax.while_loop(cond, bdy, (cur0, carg0))
        return c
    lax.fori_loop(0, _GROUPS // _W, g1, 0)

    cp_pf = pltpu.async_copy(tmax_v, shf.at[pl.ds(s * _NSEG, _NSEG)], gsem)
    cp_pi = pltpu.async_copy(warg_v, shi.at[pl.ds(s * _NSEG, _NSEG)], gsem)
    cp_z.wait()
    cp_pf.wait()
    cp_pi.wait()
    plsc.subcore_barrier()

    @pl.when(s < _NMERGE)
    def _merge():
        cps = []
        for r in range(_NS):
            src = pl.ds(r * _NSEG + s * _SEGS_PER_MTILE, _SEGS_PER_MTILE)
            dst = pl.ds(r * _SEGS_PER_MTILE, _SEGS_PER_MTILE)
            cps.append(pltpu.async_copy(shf.at[src], mrg_f.at[dst], sem))
            cps.append(pltpu.async_copy(shi.at[src], mrg_i.at[dst], sem))
        for cp in cps:
            cp.wait()
        for c8 in range(_SEGS_PER_MTILE // 16):
            av = mrg_f[pl.ds(c8 * 16, 16)]
            aa = mrg_i[pl.ds(c8 * 16, 16)]
            for r in range(1, _NS):
                bv = mrg_f[pl.ds(r * _SEGS_PER_MTILE + c8 * 16, 16)]
                ba = mrg_i[pl.ds(r * _SEGS_PER_MTILE + c8 * 16, 16)]
                take = (bv > av) | ((bv == av) & (ba < aa))
                av = jnp.where(take, bv, av)
                aa = jnp.where(take, ba, aa)
            aa = jnp.where(aa < _NCAND, aa, jnp.int32(_PAD - 1))
            widx_v[pl.ds(c8 * 16, 16)] = aa
            ones_v[pl.ds(c8 * 16, 16)] = jnp.ones((16,), jnp.float32)
        pltpu.async_copy(ones_v, out_hbm.at[widx_v], sem).wait()


_mesh = plsc.VectorSubcoreMesh(
    core_axis_name="c", subcore_axis_name="s", num_cores=1)

_call = pl.kernel(
    _body,
    out_type=jax.ShapeDtypeStruct((_PAD,), jnp.float32),
    mesh=_mesh,
    compiler_params=pltpu.CompilerParams(needs_layout_passes=False),
    scratch_types=[
        pltpu.VMEM((_CHUNK,), jnp.int32),
        pltpu.VMEM((_CHUNK,), jnp.float32),
        pltpu.VMEM((_CHUNK,), jnp.int32),
        pltpu.VMEM((_NSEG,), jnp.float32),
        pltpu.VMEM((_NSEG,), jnp.int32),
        pltpu.VMEM((_NS * _SEGS_PER_MTILE,), jnp.float32),
        pltpu.VMEM((_NS * _SEGS_PER_MTILE,), jnp.int32),
        pltpu.VMEM((_SEGS_PER_MTILE,), jnp.int32),
        pltpu.VMEM((_SEGS_PER_MTILE,), jnp.float32),
        pltpu.VMEM((_CHUNK,), jnp.float32),
        pltpu.VMEM_SHARED((_NNODES_PAD,), jnp.int32),
        pltpu.VMEM_SHARED((_NS * _NSEG,), jnp.float32),
        pltpu.VMEM_SHARED((_NS * _NSEG,), jnp.int32),
        pltpu.SemaphoreType.DMA,
        pltpu.SemaphoreType.DMA,
    ],
)


def kernel(log_probs, batch, candidate_idxs):
    batch_pad = jnp.concatenate(
        [batch, jnp.zeros((_NNODES_PAD - _NNODES,), jnp.int32)])
    winners = _call(log_probs, candidate_idxs, batch_pad)[:_NCAND]
    return (log_probs, winners)
